# baseline (speedup 1.0000x reference)
import functools

import jax
import jax.numpy as jnp
from jax import lax
from jax.experimental import pallas as pl
from jax.experimental.pallas import tpu as pltpu
from jax.experimental.pallas import tpu_sc as plsc

EMBED = 64
BATCH = 4096
HIST = 200

NC, NS = 2, 16
NW = NC * NS
B_PER_W = BATCH // NW
CHUNK = 100
CPAD = 104
CHUNKS_PER_B = HIST // CHUNK
NCHUNK_W = B_PER_W * CHUNKS_PER_B
NBUF = 4


def _pool_body(q_hbm, table_hbm, out_hbm, idx_v, bufs, out_v, sems):
    wid = lax.axis_index("s") * NC + lax.axis_index("c")
    cbase = wid * NCHUNK_W
    pltpu.sync_copy(q_hbm.at[pl.ds(cbase, NCHUNK_W)], idx_v)

    def start(cb, k):
        pltpu.async_copy(table_hbm.at[idx_v.at[cb]], bufs[k], sems[k])

    def wait(k):
        pltpu.make_async_copy(table_hbm.at[idx_v.at[0]], bufs[k], sems[k]).wait()

    for k in range(NBUF):
        start(k, k)

    def body(bb, _):
        for half in range(2):
            b = 2 * bb + half
            accs = [jnp.zeros((16,), jnp.float32) for _ in range(4)]
            for c in range(2):
                k = 2 * half + c
                cb = 4 * bb + k
                wait(k)
                for r in range(CHUNK):
                    for j in range(4):
                        accs[j] = accs[j] + bufs[k][r, pl.ds(16 * j, 16)]

                @pl.when(cb + NBUF < NCHUNK_W)
                def _():
                    start(cb + NBUF, k)

            for j in range(4):
                out_v[b, pl.ds(16 * j, 16)] = accs[j]
        return 0

    lax.fori_loop(0, B_PER_W // 2, body, 0)
    pltpu.sync_copy(out_v, out_hbm.at[pl.ds(wid * B_PER_W, B_PER_W)])


@jax.jit
def _pool(table, q8):
    mesh = plsc.VectorSubcoreMesh(core_axis_name="c", subcore_axis_name="s")
    return pl.kernel(
        _pool_body,
        out_type=jax.ShapeDtypeStruct((BATCH, EMBED), jnp.float32),
        mesh=mesh,
        scratch_types=[
            pltpu.VMEM((NCHUNK_W, CPAD), jnp.int32),
            [pltpu.VMEM((CPAD, EMBED), jnp.float32) for _ in range(NBUF)],
            pltpu.VMEM((B_PER_W, EMBED), jnp.float32),
            [pltpu.SemaphoreType.DMA for _ in range(NBUF)],
        ],
        compiler_params=pltpu.CompilerParams(use_tc_tiling_on_sc=False),
    )(q8, table)


def _mm_body(x_ref, w_ref, b_ref, o_ref):
    acc = lax.dot_general(
        x_ref[...], w_ref[...],
        dimension_numbers=(((1,), (1,)), ((), ())),
        preferred_element_type=jnp.float32,
        precision=lax.Precision.HIGHEST,
    )
    o_ref[...] = acc + b_ref[...]


BM, BN = 1024, 768


@jax.jit
def _matmul(x, w, b2d):
    n = w.shape[0]
    grid = (BATCH // BM, pl.cdiv(n, BN))
    return pl.pallas_call(
        _mm_body,
        grid=grid,
        in_specs=[
            pl.BlockSpec((BM, EMBED), lambda i, j: (i, 0)),
            pl.BlockSpec((BN, EMBED), lambda i, j: (j, 0)),
            pl.BlockSpec((1, BN), lambda i, j: (0, j)),
        ],
        out_specs=pl.BlockSpec((BM, BN), lambda i, j: (i, j)),
        out_shape=jax.ShapeDtypeStruct((BATCH, n), jnp.float32),
        compiler_params=pltpu.CompilerParams(
            dimension_semantics=("parallel", "parallel"),
        ),
    )(x, w, b2d)


def kernel(embed_table, W_out, b_out, q_input, q_lens):
    del q_lens
    q = q_input.astype(jnp.int32).reshape(BATCH, CHUNKS_PER_B, CHUNK)
    q8 = jnp.pad(q, ((0, 0), (0, 0), (0, CPAD - CHUNK))).reshape(
        BATCH * CHUNKS_PER_B, CPAD)
    pooled = _pool(embed_table, q8)
    return _matmul(pooled, W_out, b_out.reshape(1, -1))


# speedup vs baseline: 1.3291x; 1.3291x over previous
import jax
import jax.numpy as jnp
from jax import lax
from jax.experimental import pallas as pl
from jax.experimental.pallas import tpu as pltpu
from jax.experimental.pallas import tpu_sc as plsc

EMBED = 64
BATCH = 4096
HIST = 200
HPAD = 208

NC, NS = 2, 16
NW = NC * NS
B_PER_W = BATCH // NW
NG = HPAD // 16


def _pool_body(q_hbm, table_hbm, out_hbm, q_v, bufs, out_v, sems):
    wid = lax.axis_index("s") * NC + lax.axis_index("c")
    base = wid * B_PER_W
    pltpu.sync_copy(q_hbm.at[pl.ds(wid * (B_PER_W * HPAD // 128),
                                   B_PER_W * HPAD // 128)], q_v)

    def idx_vec(b, g):
        m = 13 * b + g
        return q_v[lax.shift_right_logical(m, 3),
                   pl.ds(jnp.bitwise_and(m, 7) * 16, 16)]

    def start_row(b, p):
        for g in range(NG):
            pltpu.async_copy(table_hbm.at[idx_vec(b, g)], bufs[p][g],
                             sems[p])

    def drain_row(p):
        for g in range(NG):
            pltpu.make_async_copy(table_hbm.at[q_v[0, pl.ds(0, 16)]],
                                  bufs[p][g], sems[p]).wait()

    def accum_row(p):
        a = [jnp.zeros((16,), jnp.float32) for _ in range(8)]
        for g in range(NG):
            for rr in range(16):
                h = 4 * (rr % 2)
                for half in range(2):
                    v = bufs[p][g][rr, pl.ds(16 * half, 16)]
                    lo = plsc.bitcast(lax.shift_left(v, 16), jnp.float32)
                    hi = plsc.bitcast(
                        jnp.bitwise_and(v, jnp.int32(-65536)), jnp.float32)
                    a[h + 2 * half] = a[h + 2 * half] + lo
                    a[h + 2 * half + 1] = a[h + 2 * half + 1] + hi
        return tuple(a[j] + a[4 + j] for j in range(4))

    start_row(0, 0)
    start_row(1, 1)

    def body(bb, _):
        for p in range(2):
            b = 2 * bb + p
            drain_row(p)
            accs = accum_row(p)

            @pl.when(bb < B_PER_W // 2 - 1)
            def _():
                start_row(b + 2, p)

            for j in range(4):
                out_v[b, pl.ds(16 * j, 16)] = accs[j]
        return 0

    lax.fori_loop(0, B_PER_W // 2, body, 0)
    pltpu.sync_copy(out_v, out_hbm.at[pl.ds(base, B_PER_W)])


@jax.jit
def _pool(table, q8):
    mesh = plsc.VectorSubcoreMesh(core_axis_name="c", subcore_axis_name="s")
    return pl.kernel(
        _pool_body,
        out_type=jax.ShapeDtypeStruct((BATCH, EMBED), jnp.float32),
        mesh=mesh,
        scratch_types=[
            pltpu.VMEM((B_PER_W * HPAD // 128, 128), jnp.int32),
            [[pltpu.VMEM((16, EMBED // 2), jnp.int32) for _ in range(NG)]
             for _ in range(2)],
            pltpu.VMEM((B_PER_W, EMBED), jnp.float32),
            [pltpu.SemaphoreType.DMA for _ in range(2)],
        ],
        compiler_params=pltpu.CompilerParams(use_tc_tiling_on_sc=False,
                                             needs_layout_passes=False),
    )(q8, table)


CAST_BR = 8000


def _cast_body(x_ref, o_ref):
    y = x_ref[...].astype(jnp.bfloat16)
    z = jax.lax.bitcast_convert_type(y, jnp.uint16).astype(jnp.int32)
    w = jnp.bitwise_or(z[:, 0:32], jnp.left_shift(z[:, 32:64], 16))
    h = CAST_BR // 4
    o_ref[...] = jnp.concatenate(
        [w[0:h], w[h:2 * h], w[2 * h:3 * h], w[3 * h:]], axis=1)


@jax.jit
def _cast(table):
    grid = (1000000 // CAST_BR,)
    return pl.pallas_call(
        _cast_body,
        grid=grid,
        in_specs=[pl.BlockSpec((CAST_BR, EMBED), lambda i: (i, 0))],
        out_specs=pl.BlockSpec((CAST_BR // 4, 2 * EMBED), lambda i: (i, 0)),
        out_shape=jax.ShapeDtypeStruct((250000, 2 * EMBED), jnp.int32),
        compiler_params=pltpu.CompilerParams(
            dimension_semantics=("parallel",),
        ),
    )(table)


def _mm_body(x_ref, w_ref, b_ref, o_ref):
    acc = lax.dot_general(
        x_ref[...], w_ref[...],
        dimension_numbers=(((1,), (1,)), ((), ())),
        preferred_element_type=jnp.float32,
    )
    o_ref[...] = acc + b_ref[...]


BM, BN = 1024, 768


@jax.jit
def _matmul(x, w, b2d):
    n = w.shape[0]
    grid = (BATCH // BM, pl.cdiv(n, BN))
    return pl.pallas_call(
        _mm_body,
        grid=grid,
        in_specs=[
            pl.BlockSpec((BM, EMBED), lambda i, j: (i, 0)),
            pl.BlockSpec((BN, EMBED), lambda i, j: (j, 0)),
            pl.BlockSpec((1, BN), lambda i, j: (0, j)),
        ],
        out_specs=pl.BlockSpec((BM, BN), lambda i, j: (i, j)),
        out_shape=jax.ShapeDtypeStruct((BATCH, n), jnp.float32),
        compiler_params=pltpu.CompilerParams(
            dimension_semantics=("parallel", "parallel"),
        ),
    )(x, w, b2d)


def kernel(embed_table, W_out, b_out, q_input, q_lens):
    del q_lens
    q = q_input.astype(jnp.int32)
    qm = (q // CAST_BR) * CAST_BR + 4 * (q % (CAST_BR // 4)) \
        + (q % CAST_BR) // (CAST_BR // 4)
    q8 = jnp.pad(qm, ((0, 0), (0, HPAD - HIST))).reshape(
        BATCH * HPAD // 128, 128)
    table_pk = _cast(embed_table).reshape(1000000, EMBED // 2)
    pooled = _pool(table_pk, q8)
    perm = jnp.concatenate([
        jnp.arange(0, 16), jnp.arange(32, 48),
        jnp.arange(16, 32), jnp.arange(48, 64)])
    w_perm = W_out[:, perm]
    return _matmul(pooled, w_perm, b_out.reshape(1, -1))
